# baseline (speedup 1.0000x reference)
import jax
import jax.numpy as jnp
from jax import lax
from jax.experimental import pallas as pl
from jax.experimental.pallas import tpu as pltpu
from jax.experimental.pallas import tpu_sc as plsc

VOCAB = 1000000
DIM = 64
BATCH = 4096
HIST = 200

B = BATCH * HIST
NC = 2
NS = 16
NW = NC * NS
BPW = B // NW
GATHER = 128
CG = 8
CHUNK = GATHER * CG
NCH = BPW // CHUNK


def _emb_body(x_hbm, table_hbm, out_hbm, idx_v, rows_v, gsem):
    wid = lax.axis_index("s") * NC + lax.axis_index("c")

    def chunk(c, carry):
        pltpu.sync_copy(x_hbm.at[wid, c], idx_v)
        copies = []
        for j in range(CG):
            copies.append(
                pltpu.async_copy(
                    table_hbm.at[idx_v.at[j]],
                    rows_v.at[pl.ds(j * GATHER, GATHER)],
                    gsem,
                )
            )
        for cp in copies:
            cp.wait()
        pltpu.sync_copy(
            rows_v, out_hbm.at[pl.ds(wid * BPW + c * CHUNK, CHUNK)]
        )
        return carry

    lax.fori_loop(0, NCH, chunk, 0)


@jax.jit
def _emb(x_flat, table):
    mesh = plsc.VectorSubcoreMesh(core_axis_name="c", subcore_axis_name="s")
    run = pl.kernel(
        _emb_body,
        out_type=jax.ShapeDtypeStruct((B, DIM), jnp.float32),
        mesh=mesh,
        scratch_types=[
            pltpu.VMEM((CG, GATHER), jnp.int32),
            pltpu.VMEM((CHUNK, DIM), jnp.float32),
            pltpu.SemaphoreType.DMA,
        ],
        compiler_params=pltpu.CompilerParams(use_tc_tiling_on_sc=False),
    )
    return run(x_flat, table)


def kernel(x, table):
    x_flat = x.reshape(NW, NCH, CG, GATHER).astype(jnp.int32)
    out = _emb(x_flat, table)
    return out.reshape(BATCH, HIST, DIM)


# speedup vs baseline: 1.0164x; 1.0164x over previous
import jax
import jax.numpy as jnp
from jax import lax
from jax.experimental import pallas as pl
from jax.experimental.pallas import tpu as pltpu
from jax.experimental.pallas import tpu_sc as plsc

VOCAB = 1000000
DIM = 64
BATCH = 4096
HIST = 200

B = BATCH * HIST
NC = 2
NS = 16
NW = NC * NS
BPW = B // NW
GATHER = 128
CG = 4
CHUNK = GATHER * CG
NCH = BPW // CHUNK
NBLK = BPW // GATHER
ROW_BYTES = CHUNK * DIM * 4


def _emb_body(x_hbm, table_hbm, out_hbm, idx_v, rows0, rows1, g0, g1, o0, o1):
    wid = lax.axis_index("s") * NC + lax.axis_index("c")
    base = wid * BPW

    pltpu.sync_copy(x_hbm.at[wid], idx_v)

    def fire(c, rows, sem):
        for j in range(CG):
            pltpu.async_copy(
                table_hbm.at[idx_v.at[c * CG + j]],
                rows.at[pl.ds(j * GATHER, GATHER)],
                sem,
            )

    def drain_gather(rows, sem):
        pltpu.make_async_copy(out_hbm.at[pl.ds(0, CHUNK)], rows, sem).wait()

    def writeback(c, rows, sem):
        pltpu.async_copy(rows, out_hbm.at[pl.ds(base + c * CHUNK, CHUNK)], sem)

    def drain_wb(rows, sem):
        pltpu.make_async_copy(rows, out_hbm.at[pl.ds(0, CHUNK)], sem).wait()

    fire(0, rows0, g0)
    fire(1, rows1, g1)
    drain_gather(rows0, g0)
    writeback(0, rows0, o0)

    def step(s, carry):
        c = 2 * s + 2
        d = c + 1
        drain_wb(rows0, o0)
        fire(c, rows0, g0)
        drain_gather(rows1, g1)
        writeback(c - 1, rows1, o1)
        drain_wb(rows1, o1)
        fire(d, rows1, g1)
        drain_gather(rows0, g0)
        writeback(c, rows0, o0)
        return carry

    lax.fori_loop(0, (NCH - 2) // 2, step, 0)

    drain_gather(rows1, g1)
    writeback(NCH - 1, rows1, o1)
    drain_wb(rows0, o0)
    drain_wb(rows1, o1)


@jax.jit
def _emb(x_blocks, table):
    mesh = plsc.VectorSubcoreMesh(core_axis_name="c", subcore_axis_name="s")
    run = pl.kernel(
        _emb_body,
        out_type=jax.ShapeDtypeStruct((B, DIM), jnp.float32),
        mesh=mesh,
        scratch_types=[
            pltpu.VMEM((NBLK, GATHER), jnp.int32),
            pltpu.VMEM((CHUNK, DIM), jnp.float32),
            pltpu.VMEM((CHUNK, DIM), jnp.float32),
            pltpu.SemaphoreType.DMA,
            pltpu.SemaphoreType.DMA,
            pltpu.SemaphoreType.DMA,
            pltpu.SemaphoreType.DMA,
        ],
        compiler_params=pltpu.CompilerParams(use_tc_tiling_on_sc=False),
    )
    return run(x_blocks, table)


def kernel(x, table):
    x_blocks = x.reshape(NW, NBLK, GATHER).astype(jnp.int32)
    out = _emb(x_blocks, table)
    return out.reshape(BATCH, HIST, DIM)
